# baseline (speedup 1.0000x reference)
import functools

import jax
import jax.numpy as jnp
from jax import lax
from jax.experimental import pallas as pl
from jax.experimental.pallas import tpu as pltpu
from jax.experimental.pallas import tpu_sc as plsc

_NC = 2
_NS = 16
_CH = 128
_NB = 3
_SNB = 2


def _pack_bf16_pair(m, DP):
    u = jax.lax.bitcast_convert_type(m, jnp.uint32)
    rnd = (u + ((u >> 16) & jnp.uint32(1)) + jnp.uint32(0x7FFF)) \
        & jnp.uint32(0xFFFF0000)
    packed = rnd[:, :DP] | (rnd[:, DP:] >> 16)
    return jax.lax.bitcast_convert_type(packed, jnp.int32)


def _unpack_bf16_pair(p):
    u = jax.lax.bitcast_convert_type(p, jnp.uint32)
    a = jax.lax.bitcast_convert_type(u & jnp.uint32(0xFFFF0000), jnp.float32)
    b = jax.lax.bitcast_convert_type(u << 16, jnp.float32)
    return a, b


def _proj_body(x_ref, wd_ref, ws_ref, pd_ref, ps_ref):
    xb = x_ref[...]
    dp = pd_ref.shape[-1]
    pd_ref[...] = _pack_bf16_pair(
        jnp.dot(xb, wd_ref[...], preferred_element_type=jnp.float32), dp)
    ps_ref[...] = _pack_bf16_pair(
        jnp.dot(xb, ws_ref[...], preferred_element_type=jnp.float32), dp)


def _proj(N, D, BN):
    return pl.pallas_call(
        _proj_body,
        grid=(N // BN,),
        in_specs=[
            pl.BlockSpec((BN, D), lambda i: (i, 0)),
            pl.BlockSpec((D, D), lambda i: (0, 0)),
            pl.BlockSpec((D, D), lambda i: (0, 0)),
        ],
        out_specs=[
            pl.BlockSpec((BN, D // 2), lambda i: (i, 0)),
            pl.BlockSpec((BN, D // 2), lambda i: (i, 0)),
        ],
        out_shape=[
            jax.ShapeDtypeStruct((N, D // 2), jnp.int32),
            jax.ShapeDtypeStruct((N, D // 2), jnp.int32),
        ],
    )


def _sc_gather(N, D, ES, ebase):
    DP = D // 2
    nw = _NC * _NS
    ew = ES // nw
    assert ES % nw == 0 and ew % 8 == 0 and ebase % 8 == 0 and ew >= _CH
    nch = -(-ew // _CH)

    mesh = plsc.VectorSubcoreMesh(core_axis_name="c", subcore_axis_name="s")

    @functools.partial(
        pl.kernel,
        mesh=mesh,
        compiler_params=pltpu.CompilerParams(use_tc_tiling_on_sc=False),
        out_type=[
            jax.ShapeDtypeStruct((ES, DP), jnp.int32),
            jax.ShapeDtypeStruct((ES, DP), jnp.int32),
        ],
        scratch_types=(
            [pltpu.VMEM((_CH,), jnp.int32) for _ in range(2 * _NB)]
            + [pltpu.VMEM((_CH, DP), jnp.int32) for _ in range(2 * _NB)]
            + [pltpu.SemaphoreType.DMA for _ in range(2 * _NB)]
        ),
    )
    def gather_k(pd_hbm, ps_hbm, dst_hbm, src_hbm, gd_hbm, gs_hbm, *scr):
        c = lax.axis_index("c")
        s = lax.axis_index("s")
        wid = s * _NC + c
        base_w = wid * ew
        idxs = scr[:2 * _NB]
        rows = scr[2 * _NB:4 * _NB]
        sems = scr[4 * _NB:]
        bufs = tuple(
            (idxs[2 * b], idxs[2 * b + 1], rows[2 * b], rows[2 * b + 1],
             sems[2 * b], sems[2 * b + 1])
            for b in range(_NB))
        la = _NB - 1

        def off(ch):
            return base_w + jnp.minimum(ch * _CH, ew - _CH)

        def fire(ch, b):
            dstv, srcv, rdv, rsv, gsem, _ = bufs[b]
            o = off(ch)
            pltpu.sync_copy(dst_hbm.at[pl.ds(ebase + o, _CH)], dstv)
            pltpu.sync_copy(src_hbm.at[pl.ds(ebase + o, _CH)], srcv)
            pltpu.async_copy(pd_hbm.at[dstv], rdv, gsem)
            pltpu.async_copy(ps_hbm.at[srcv], rsv, gsem)

        def drain_and_write(ch, b):
            dstv, srcv, rdv, rsv, gsem, wsem = bufs[b]
            o = off(ch)
            pltpu.make_async_copy(pd_hbm.at[dstv], rdv, gsem).wait()
            pltpu.make_async_copy(ps_hbm.at[srcv], rsv, gsem).wait()
            pltpu.async_copy(rdv, gd_hbm.at[pl.ds(o, _CH)], wsem)
            pltpu.async_copy(rsv, gs_hbm.at[pl.ds(o, _CH)], wsem)

        def wait_writes(ch, b):
            _, _, rdv, rsv, _, wsem = bufs[b]
            o = off(ch)
            pltpu.make_async_copy(rdv, gd_hbm.at[pl.ds(o, _CH)], wsem).wait()
            pltpu.make_async_copy(rsv, gs_hbm.at[pl.ds(o, _CH)], wsem).wait()

        for p in range(min(la, nch)):
            fire(p, p)

        @pl.loop(0, _NB * (-(-nch // _NB)), step=_NB)
        def _blk(i):
            for b in range(_NB):
                ch = i + b
                nxt = ch + la
                fb = (b + la) % _NB

                @pl.when(nxt < nch)
                def _():
                    @pl.when(nxt >= _NB)
                    def _():
                        wait_writes(nxt - _NB, fb)
                    fire(nxt, fb)

                @pl.when(ch < nch)
                def _():
                    drain_and_write(ch, b)

        for q in range(max(0, nch - _NB), nch):
            wait_writes(q, q % _NB)

    return gather_k


def _edge_body(gd_ref, gs_ref, amf_lo, ea_lo, amf_hi, ea_hi,
               w1a_ref, w1b_ref, b1a_ref, b1b_ref,
               w2ae_ref, w2be_ref, w2ao_ref, w2bo_ref, w2Ae_ref, w2Ao_ref,
               b2_ref, mev_ref, mod_ref):
    ad, bd = _unpack_bf16_pair(gd_ref[...])
    asrc, bsrc = _unpack_bf16_pair(gs_ref[...])
    ap = jnp.concatenate(
        [amf_lo[...], ea_lo[...], amf_hi[...], ea_hi[...]], axis=-1)

    def mm(x, w_ref):
        return jnp.dot(x, w_ref[...], preferred_element_type=jnp.float32)

    ha = jax.nn.silu(ad + asrc + mm(ap, w1a_ref) + b1a_ref[...])
    hb = jax.nn.silu(bd + bsrc + mm(ap, w1b_ref) + b1b_ref[...])
    ha16 = ha.astype(jnp.bfloat16)
    hb16 = hb.astype(jnp.bfloat16)
    mev_ref[...] = jax.nn.silu(
        mm(ha16, w2ae_ref) + mm(hb16, w2be_ref) + mm(ap, w2Ae_ref)
        + b2_ref[...])
    mod_ref[...] = jax.nn.silu(
        mm(ha16, w2ao_ref) + mm(hb16, w2bo_ref) + mm(ap, w2Ao_ref)
        + b2_ref[...])


def _edge_mlp(ES, D, DA, BE):
    BE2 = BE // 2
    hoff = (ES // 2) // BE2
    full = lambda shape: pl.BlockSpec(shape, lambda i: (0, 0))
    return pl.pallas_call(
        _edge_body,
        grid=(ES // BE,),
        in_specs=[
            pl.BlockSpec((BE2, D), lambda i: (i, 0)),
            pl.BlockSpec((BE2, D), lambda i: (i, 0)),
            pl.BlockSpec((BE2, DA), lambda i: (i, 0)),
            pl.BlockSpec((BE2, DA), lambda i: (i, 0)),
            pl.BlockSpec((BE2, DA), lambda i: (i + hoff, 0)),
            pl.BlockSpec((BE2, DA), lambda i: (i + hoff, 0)),
            full((4 * DA, D)), full((4 * DA, D)),
            full((1, D)), full((1, D)),
            full((D, D)), full((D, D)),
            full((D, D)), full((D, D)),
            full((4 * DA, D)), full((4 * DA, D)),
            full((1, D)),
        ],
        out_specs=[
            pl.BlockSpec((BE2, D), lambda i: (i, 0)),
            pl.BlockSpec((BE2, D), lambda i: (i, 0)),
        ],
        out_shape=[
            jax.ShapeDtypeStruct((ES // 2, D), jnp.float32),
            jax.ShapeDtypeStruct((ES // 2, D), jnp.float32),
        ],
    )


def _sc_scatter(N, E, D, K):
    nw = _NC * _NS
    ES = E // K
    ews = ES // nw
    ch_s = 120
    assert E % K == 0 and ES % nw == 0 and ews % 8 == 0 and ch_s % 8 == 0
    nfull = ews // ch_s
    tail = ews - nfull * ch_s
    assert tail % 8 == 0
    npad = -(-N // (8 * _NS)) * (8 * _NS)
    rps = npad // _NS

    mesh = plsc.VectorSubcoreMesh(core_axis_name="c", subcore_axis_name="s")

    scratch = (
        [pltpu.VMEM((ch_s,), jnp.int32) for _ in range(_SNB)]
        + [pltpu.VMEM((ch_s, D), jnp.float32) for _ in range(_SNB)]
        + [pltpu.SemaphoreType.DMA for _ in range(2 * _SNB)]
        + [pltpu.VMEM_SHARED((npad, D), jnp.float32)]
    )
    if tail:
        scratch += [
            pltpu.VMEM((tail,), jnp.int32),
            pltpu.VMEM((tail, D), jnp.float32),
        ]

    @functools.partial(
        pl.kernel,
        mesh=mesh,
        out_type=jax.ShapeDtypeStruct((_NC * npad, D), jnp.float32),
        scratch_types=scratch,
    )
    def scatter_k(*args):
        ms = args[:K]
        dst_hbm, zeros_hbm, out_hbm = args[K:K + 3]
        scr = args[K + 3:]
        c = lax.axis_index("c")
        s = lax.axis_index("s")
        wid = s * _NC + c
        base_w = wid * ews
        idxs = scr[:_SNB]
        mrows = scr[_SNB:2 * _SNB]
        sems = scr[2 * _SNB:4 * _SNB]
        acc = scr[4 * _SNB]
        tl = scr[4 * _SNB + 1:]
        bufs = tuple((idxs[b], mrows[b], sems[2 * b], sems[2 * b + 1])
                     for b in range(_SNB))
        la = _SNB - 1

        pltpu.sync_copy(zeros_hbm.at[pl.ds(s * rps, rps)],
                        acc.at[pl.ds(s * rps, rps)])
        plsc.subcore_barrier()

        def fire_loads(m_hbm, sbase, ch, b):
            dstv, mv, lsem, _ = bufs[b]
            o = base_w + ch * ch_s
            pltpu.async_copy(dst_hbm.at[pl.ds(sbase + o, ch_s)], dstv, lsem)
            pltpu.async_copy(m_hbm.at[pl.ds(o, ch_s)], mv, lsem)

        def fire_scatter(m_hbm, sbase, ch, b):
            dstv, mv, lsem, ssem = bufs[b]
            o = base_w + ch * ch_s
            pltpu.make_async_copy(
                dst_hbm.at[pl.ds(sbase + o, ch_s)], dstv, lsem).wait()
            pltpu.make_async_copy(m_hbm.at[pl.ds(o, ch_s)], mv, lsem).wait()
            pltpu.async_copy(mv, acc.at[dstv], ssem, add=True)

        def wait_scatter(b):
            dstv, mv, _, ssem = bufs[b]
            pltpu.make_async_copy(mv, acc.at[dstv], ssem).wait()

        for k in range(K):
            m_hbm = ms[k]
            sbase = k * ES
            for p in range(min(la, nfull)):
                fire_loads(m_hbm, sbase, p, p)

            @pl.loop(0, _SNB * (-(-nfull // _SNB)), step=_SNB)
            def _blk(i):
                for b in range(_SNB):
                    ch = i + b
                    nxt = ch + la
                    fb = (b + la) % _SNB

                    @pl.when(nxt < nfull)
                    def _():
                        @pl.when(nxt >= _SNB)
                        def _():
                            wait_scatter(fb)
                        fire_loads(m_hbm, sbase, nxt, fb)

                    @pl.when(ch < nfull)
                    def _():
                        fire_scatter(m_hbm, sbase, ch, b)

            for q in range(max(0, nfull - _SNB), nfull):
                wait_scatter(q % _SNB)

            if tail:
                dstv_t, mv_t = tl
                o = base_w + nfull * ch_s
                pltpu.sync_copy(dst_hbm.at[pl.ds(sbase + o, tail)], dstv_t)
                pltpu.sync_copy(m_hbm.at[pl.ds(o, tail)], mv_t)
                pltpu.sync_copy(mv_t, acc.at[dstv_t], add=True)

        plsc.subcore_barrier()
        pltpu.sync_copy(acc.at[pl.ds(s * rps, rps)],
                        out_hbm.at[pl.ds(c * npad + s * rps, rps)])

    return scatter_k


def _update_body(x_ref, p_ref, anf_ref, na_ref, wu1_ref, bu1_ref,
                 wu2_ref, bu2_ref, o_ref):
    na = na_ref[...]
    agg = p_ref[0] + p_ref[1]
    u_in = jnp.concatenate([x_ref[...], agg, anf_ref[...], na], axis=-1)
    u = jax.nn.silu(jnp.dot(u_in, wu1_ref[...],
                            preferred_element_type=jnp.float32) + bu1_ref[...])
    o_ref[...] = jnp.dot(jnp.concatenate([u, na], axis=-1), wu2_ref[...],
                         preferred_element_type=jnp.float32) + bu2_ref[...]


def _update(N, D, DA, BN):
    return pl.pallas_call(
        _update_body,
        grid=(N // BN,),
        in_specs=[
            pl.BlockSpec((BN, D), lambda i: (i, 0)),
            pl.BlockSpec((_NC, BN, D), lambda i: (0, i, 0)),
            pl.BlockSpec((BN, DA), lambda i: (i, 0)),
            pl.BlockSpec((BN, DA), lambda i: (i, 0)),
            pl.BlockSpec((2 * D + 2 * DA, D), lambda i: (0, 0)),
            pl.BlockSpec((1, D), lambda i: (0, 0)),
            pl.BlockSpec((D + DA, D), lambda i: (0, 0)),
            pl.BlockSpec((1, D), lambda i: (0, 0)),
        ],
        out_specs=pl.BlockSpec((BN, D), lambda i: (i, 0)),
        out_shape=jax.ShapeDtypeStruct((N, D), jnp.float32),
    )


def kernel(x, edge_index, edge_attr, node_attr, batch,
           additional_message_features, additional_node_features,
           W_m1, b_m1, W_m2, b_m2, W_u1, b_u1, W_u2, b_u2):
    del batch
    N, D = x.shape
    E, DA = edge_attr.shape
    DH = D // 2
    src = edge_index[0]
    dst = edge_index[1]

    srcp = jnp.stack([src[:E // 2], src[E // 2:]], axis=1).reshape(E)
    dstp = jnp.stack([dst[:E // 2], dst[E // 2:]], axis=1).reshape(E)

    pd, ps = _proj(N, D, 2000)(x, W_m1[:D], W_m1[D:2 * D])
    gd, gs = _sc_gather(N, D, E, 0)(pd, ps, dstp, srcp)
    gd = gd.reshape(E // 2, D)
    gs = gs.reshape(E // 2, D)

    w1t = W_m1[2 * D:]
    z = jnp.zeros((2 * DA, DH), jnp.float32)
    w1a = jnp.block([[w1t[:, :DH], z], [z, w1t[:, :DH]]])
    w1b = jnp.block([[w1t[:, DH:], z], [z, w1t[:, DH:]]])
    b1a = jnp.tile(b_m1[:DH], 2).reshape(1, D)
    b1b = jnp.tile(b_m1[DH:], 2).reshape(1, D)
    w2h = W_m2[:D]
    w2ea = W_m2[D:]
    zh = jnp.zeros((DH, D), jnp.float32)
    w2ae = jnp.concatenate([w2h[:DH], zh]).astype(jnp.bfloat16)
    w2be = jnp.concatenate([w2h[DH:], zh]).astype(jnp.bfloat16)
    w2ao = jnp.concatenate([zh, w2h[:DH]]).astype(jnp.bfloat16)
    w2bo = jnp.concatenate([zh, w2h[DH:]]).astype(jnp.bfloat16)
    za = jnp.zeros((DA, D), jnp.float32)
    w2Ae = jnp.concatenate([za, w2ea, za, za])
    w2Ao = jnp.concatenate([za, za, za, w2ea])
    b2 = b_m2.reshape(1, D)

    m_lo, m_hi = _edge_mlp(E, D, DA, 2000)(
        gd, gs, additional_message_features, edge_attr,
        additional_message_features, edge_attr, w1a, w1b, b1a, b1b,
        w2ae, w2be, w2ao, w2bo, w2Ae, w2Ao, b2)

    npad = -(-N // (8 * _NS)) * (8 * _NS)
    parts = _sc_scatter(N, E, D, 2)(
        m_lo, m_hi, dst, jnp.zeros((npad, D), jnp.float32))
    p = parts.reshape(_NC, npad, D)
    out = _update(N, D, DA, 2000)(
        x, p, additional_node_features, node_attr,
        W_u1, b_u1.reshape(1, D), W_u2, b_u2.reshape(1, D))
    return out


# speedup vs baseline: 1.0012x; 1.0012x over previous
import functools

import jax
import jax.numpy as jnp
from jax import lax
from jax.experimental import pallas as pl
from jax.experimental.pallas import tpu as pltpu
from jax.experimental.pallas import tpu_sc as plsc

_NC = 2
_NS = 16
_CH = 128
_NB = 3
_SNB = 2


def _pack_bf16_pair(m, DP):
    u = jax.lax.bitcast_convert_type(m, jnp.uint32)
    rnd = (u + ((u >> 16) & jnp.uint32(1)) + jnp.uint32(0x7FFF)) \
        & jnp.uint32(0xFFFF0000)
    packed = rnd[:, :DP] | (rnd[:, DP:] >> 16)
    return jax.lax.bitcast_convert_type(packed, jnp.int32)


def _unpack_bf16_pair(p):
    u = jax.lax.bitcast_convert_type(p, jnp.uint32)
    a = jax.lax.bitcast_convert_type(u & jnp.uint32(0xFFFF0000), jnp.float32)
    b = jax.lax.bitcast_convert_type(u << 16, jnp.float32)
    return a, b


def _proj_body(x_ref, wd_ref, ws_ref, pd_ref, ps_ref):
    xb = x_ref[...]
    dp = pd_ref.shape[-1]
    pd_ref[...] = _pack_bf16_pair(
        jnp.dot(xb, wd_ref[...], preferred_element_type=jnp.float32), dp)
    ps_ref[...] = _pack_bf16_pair(
        jnp.dot(xb, ws_ref[...], preferred_element_type=jnp.float32), dp)


def _proj(N, D, BN):
    return pl.pallas_call(
        _proj_body,
        grid=(N // BN,),
        in_specs=[
            pl.BlockSpec((BN, D), lambda i: (i, 0)),
            pl.BlockSpec((D, D), lambda i: (0, 0)),
            pl.BlockSpec((D, D), lambda i: (0, 0)),
        ],
        out_specs=[
            pl.BlockSpec((BN, D // 2), lambda i: (i, 0)),
            pl.BlockSpec((BN, D // 2), lambda i: (i, 0)),
        ],
        out_shape=[
            jax.ShapeDtypeStruct((N, D // 2), jnp.int32),
            jax.ShapeDtypeStruct((N, D // 2), jnp.int32),
        ],
    )


def _sc_gather(N, D, ES, ebase):
    DP = D // 2
    nw = _NC * _NS
    ew = ES // nw
    assert ES % nw == 0 and ew % 8 == 0 and ebase % 8 == 0 and ew >= _CH
    nch = -(-ew // _CH)

    mesh = plsc.VectorSubcoreMesh(core_axis_name="c", subcore_axis_name="s")

    @functools.partial(
        pl.kernel,
        mesh=mesh,
        compiler_params=pltpu.CompilerParams(use_tc_tiling_on_sc=False),
        out_type=[
            jax.ShapeDtypeStruct((ES, DP), jnp.int32),
            jax.ShapeDtypeStruct((ES, DP), jnp.int32),
        ],
        scratch_types=(
            [pltpu.VMEM((_CH,), jnp.int32) for _ in range(2 * _NB)]
            + [pltpu.VMEM((_CH, DP), jnp.int32) for _ in range(2 * _NB)]
            + [pltpu.SemaphoreType.DMA for _ in range(2 * _NB)]
        ),
    )
    def gather_k(pd_hbm, ps_hbm, dst_hbm, src_hbm, gd_hbm, gs_hbm, *scr):
        c = lax.axis_index("c")
        s = lax.axis_index("s")
        wid = s * _NC + c
        base_w = wid * ew
        idxs = scr[:2 * _NB]
        rows = scr[2 * _NB:4 * _NB]
        sems = scr[4 * _NB:]
        bufs = tuple(
            (idxs[2 * b], idxs[2 * b + 1], rows[2 * b], rows[2 * b + 1],
             sems[2 * b], sems[2 * b + 1])
            for b in range(_NB))
        la = _NB - 1

        def off(ch):
            return base_w + jnp.minimum(ch * _CH, ew - _CH)

        def fire(ch, b):
            dstv, srcv, rdv, rsv, gsem, _ = bufs[b]
            o = off(ch)
            pltpu.sync_copy(dst_hbm.at[pl.ds(ebase + o, _CH)], dstv)
            pltpu.sync_copy(src_hbm.at[pl.ds(ebase + o, _CH)], srcv)
            pltpu.async_copy(pd_hbm.at[dstv], rdv, gsem)
            pltpu.async_copy(ps_hbm.at[srcv], rsv, gsem)

        def drain_and_write(ch, b):
            dstv, srcv, rdv, rsv, gsem, wsem = bufs[b]
            o = off(ch)
            pltpu.make_async_copy(pd_hbm.at[dstv], rdv, gsem).wait()
            pltpu.make_async_copy(ps_hbm.at[srcv], rsv, gsem).wait()
            pltpu.async_copy(rdv, gd_hbm.at[pl.ds(o, _CH)], wsem)
            pltpu.async_copy(rsv, gs_hbm.at[pl.ds(o, _CH)], wsem)

        def wait_writes(ch, b):
            _, _, rdv, rsv, _, wsem = bufs[b]
            o = off(ch)
            pltpu.make_async_copy(rdv, gd_hbm.at[pl.ds(o, _CH)], wsem).wait()
            pltpu.make_async_copy(rsv, gs_hbm.at[pl.ds(o, _CH)], wsem).wait()

        for p in range(min(la, nch)):
            fire(p, p)

        @pl.loop(0, _NB * (-(-nch // _NB)), step=_NB)
        def _blk(i):
            for b in range(_NB):
                ch = i + b
                nxt = ch + la
                fb = (b + la) % _NB

                @pl.when(nxt < nch)
                def _():
                    @pl.when(nxt >= _NB)
                    def _():
                        wait_writes(nxt - _NB, fb)
                    fire(nxt, fb)

                @pl.when(ch < nch)
                def _():
                    drain_and_write(ch, b)

        for q in range(max(0, nch - _NB), nch):
            wait_writes(q, q % _NB)

    return gather_k


def _edge_body(gd_ref, gs_ref, amf_lo, ea_lo, amf_hi, ea_hi,
               w1a_ref, w1b_ref, b1a_ref, b1b_ref,
               w2ae_ref, w2be_ref, w2ao_ref, w2bo_ref, w2Ae_ref, w2Ao_ref,
               b2_ref, mev_ref, mod_ref):
    ad, bd = _unpack_bf16_pair(gd_ref[...])
    asrc, bsrc = _unpack_bf16_pair(gs_ref[...])
    ap = jnp.concatenate(
        [amf_lo[...], ea_lo[...], amf_hi[...], ea_hi[...]], axis=-1)

    def mm(x, w_ref):
        return jnp.dot(x, w_ref[...], preferred_element_type=jnp.float32)

    ha = jax.nn.silu(ad + asrc + mm(ap, w1a_ref) + b1a_ref[...])
    hb = jax.nn.silu(bd + bsrc + mm(ap, w1b_ref) + b1b_ref[...])
    mev_ref[...] = jax.nn.silu(
        mm(ha, w2ae_ref) + mm(hb, w2be_ref) + mm(ap, w2Ae_ref)
        + b2_ref[...])
    mod_ref[...] = jax.nn.silu(
        mm(ha, w2ao_ref) + mm(hb, w2bo_ref) + mm(ap, w2Ao_ref)
        + b2_ref[...])


def _edge_mlp(ES, D, DA, BE):
    BE2 = BE // 2
    hoff = (ES // 2) // BE2
    full = lambda shape: pl.BlockSpec(shape, lambda i: (0, 0))
    return pl.pallas_call(
        _edge_body,
        grid=(ES // BE,),
        in_specs=[
            pl.BlockSpec((BE2, D), lambda i: (i, 0)),
            pl.BlockSpec((BE2, D), lambda i: (i, 0)),
            pl.BlockSpec((BE2, DA), lambda i: (i, 0)),
            pl.BlockSpec((BE2, DA), lambda i: (i, 0)),
            pl.BlockSpec((BE2, DA), lambda i: (i + hoff, 0)),
            pl.BlockSpec((BE2, DA), lambda i: (i + hoff, 0)),
            full((4 * DA, D)), full((4 * DA, D)),
            full((1, D)), full((1, D)),
            full((D, D)), full((D, D)),
            full((D, D)), full((D, D)),
            full((4 * DA, D)), full((4 * DA, D)),
            full((1, D)),
        ],
        out_specs=[
            pl.BlockSpec((BE2, D), lambda i: (i, 0)),
            pl.BlockSpec((BE2, D), lambda i: (i, 0)),
        ],
        out_shape=[
            jax.ShapeDtypeStruct((ES // 2, D), jnp.float32),
            jax.ShapeDtypeStruct((ES // 2, D), jnp.float32),
        ],
    )


def _sc_scatter(N, E, D, K):
    nw = _NC * _NS
    ES = E // K
    ews = ES // nw
    ch_s = 120
    assert E % K == 0 and ES % nw == 0 and ews % 8 == 0 and ch_s % 8 == 0
    nfull = ews // ch_s
    tail = ews - nfull * ch_s
    assert tail % 8 == 0
    npad = -(-N // (8 * _NS)) * (8 * _NS)
    rps = npad // _NS

    mesh = plsc.VectorSubcoreMesh(core_axis_name="c", subcore_axis_name="s")

    scratch = (
        [pltpu.VMEM((ch_s,), jnp.int32) for _ in range(_SNB)]
        + [pltpu.VMEM((ch_s, D), jnp.float32) for _ in range(_SNB)]
        + [pltpu.SemaphoreType.DMA for _ in range(2 * _SNB)]
        + [pltpu.VMEM_SHARED((npad, D), jnp.float32)]
    )
    if tail:
        scratch += [
            pltpu.VMEM((tail,), jnp.int32),
            pltpu.VMEM((tail, D), jnp.float32),
        ]

    @functools.partial(
        pl.kernel,
        mesh=mesh,
        out_type=jax.ShapeDtypeStruct((_NC * npad, D), jnp.float32),
        scratch_types=scratch,
    )
    def scatter_k(*args):
        ms = args[:K]
        dst_hbm, zeros_hbm, out_hbm = args[K:K + 3]
        scr = args[K + 3:]
        c = lax.axis_index("c")
        s = lax.axis_index("s")
        wid = s * _NC + c
        base_w = wid * ews
        idxs = scr[:_SNB]
        mrows = scr[_SNB:2 * _SNB]
        sems = scr[2 * _SNB:4 * _SNB]
        acc = scr[4 * _SNB]
        tl = scr[4 * _SNB + 1:]
        bufs = tuple((idxs[b], mrows[b], sems[2 * b], sems[2 * b + 1])
                     for b in range(_SNB))
        la = _SNB - 1

        pltpu.sync_copy(zeros_hbm.at[pl.ds(s * rps, rps)],
                        acc.at[pl.ds(s * rps, rps)])
        plsc.subcore_barrier()

        def fire_loads(m_hbm, sbase, ch, b):
            dstv, mv, lsem, _ = bufs[b]
            o = base_w + ch * ch_s
            pltpu.async_copy(dst_hbm.at[pl.ds(sbase + o, ch_s)], dstv, lsem)
            pltpu.async_copy(m_hbm.at[pl.ds(o, ch_s)], mv, lsem)

        def fire_scatter(m_hbm, sbase, ch, b):
            dstv, mv, lsem, ssem = bufs[b]
            o = base_w + ch * ch_s
            pltpu.make_async_copy(
                dst_hbm.at[pl.ds(sbase + o, ch_s)], dstv, lsem).wait()
            pltpu.make_async_copy(m_hbm.at[pl.ds(o, ch_s)], mv, lsem).wait()
            pltpu.async_copy(mv, acc.at[dstv], ssem, add=True)

        def wait_scatter(b):
            dstv, mv, _, ssem = bufs[b]
            pltpu.make_async_copy(mv, acc.at[dstv], ssem).wait()

        for k in range(K):
            m_hbm = ms[k]
            sbase = k * ES
            for p in range(min(la, nfull)):
                fire_loads(m_hbm, sbase, p, p)

            @pl.loop(0, _SNB * (-(-nfull // _SNB)), step=_SNB)
            def _blk(i):
                for b in range(_SNB):
                    ch = i + b
                    nxt = ch + la
                    fb = (b + la) % _SNB

                    @pl.when(nxt < nfull)
                    def _():
                        @pl.when(nxt >= _SNB)
                        def _():
                            wait_scatter(fb)
                        fire_loads(m_hbm, sbase, nxt, fb)

                    @pl.when(ch < nfull)
                    def _():
                        fire_scatter(m_hbm, sbase, ch, b)

            for q in range(max(0, nfull - _SNB), nfull):
                wait_scatter(q % _SNB)

            if tail:
                dstv_t, mv_t = tl
                o = base_w + nfull * ch_s
                pltpu.sync_copy(dst_hbm.at[pl.ds(sbase + o, tail)], dstv_t)
                pltpu.sync_copy(m_hbm.at[pl.ds(o, tail)], mv_t)
                pltpu.sync_copy(mv_t, acc.at[dstv_t], add=True)

        plsc.subcore_barrier()
        pltpu.sync_copy(acc.at[pl.ds(s * rps, rps)],
                        out_hbm.at[pl.ds(c * npad + s * rps, rps)])

    return scatter_k


def _update_body(x_ref, p_ref, anf_ref, na_ref, wu1_ref, bu1_ref,
                 wu2_ref, bu2_ref, o_ref):
    na = na_ref[...]
    agg = p_ref[0] + p_ref[1]
    u_in = jnp.concatenate([x_ref[...], agg, anf_ref[...], na], axis=-1)
    u = jax.nn.silu(jnp.dot(u_in, wu1_ref[...],
                            preferred_element_type=jnp.float32) + bu1_ref[...])
    o_ref[...] = jnp.dot(jnp.concatenate([u, na], axis=-1), wu2_ref[...],
                         preferred_element_type=jnp.float32) + bu2_ref[...]


def _update(N, D, DA, BN):
    return pl.pallas_call(
        _update_body,
        grid=(N // BN,),
        in_specs=[
            pl.BlockSpec((BN, D), lambda i: (i, 0)),
            pl.BlockSpec((_NC, BN, D), lambda i: (0, i, 0)),
            pl.BlockSpec((BN, DA), lambda i: (i, 0)),
            pl.BlockSpec((BN, DA), lambda i: (i, 0)),
            pl.BlockSpec((2 * D + 2 * DA, D), lambda i: (0, 0)),
            pl.BlockSpec((1, D), lambda i: (0, 0)),
            pl.BlockSpec((D + DA, D), lambda i: (0, 0)),
            pl.BlockSpec((1, D), lambda i: (0, 0)),
        ],
        out_specs=pl.BlockSpec((BN, D), lambda i: (i, 0)),
        out_shape=jax.ShapeDtypeStruct((N, D), jnp.float32),
    )


def kernel(x, edge_index, edge_attr, node_attr, batch,
           additional_message_features, additional_node_features,
           W_m1, b_m1, W_m2, b_m2, W_u1, b_u1, W_u2, b_u2):
    del batch
    N, D = x.shape
    E, DA = edge_attr.shape
    DH = D // 2
    src = edge_index[0]
    dst = edge_index[1]

    srcp = jnp.stack([src[:E // 2], src[E // 2:]], axis=1).reshape(E)
    dstp = jnp.stack([dst[:E // 2], dst[E // 2:]], axis=1).reshape(E)

    pd, ps = _proj(N, D, 2000)(x, W_m1[:D], W_m1[D:2 * D])
    gd, gs = _sc_gather(N, D, E, 0)(pd, ps, dstp, srcp)
    gd = gd.reshape(E // 2, D)
    gs = gs.reshape(E // 2, D)

    w1t = W_m1[2 * D:]
    z = jnp.zeros((2 * DA, DH), jnp.float32)
    w1a = jnp.block([[w1t[:, :DH], z], [z, w1t[:, :DH]]])
    w1b = jnp.block([[w1t[:, DH:], z], [z, w1t[:, DH:]]])
    b1a = jnp.tile(b_m1[:DH], 2).reshape(1, D)
    b1b = jnp.tile(b_m1[DH:], 2).reshape(1, D)
    w2h = W_m2[:D]
    w2ea = W_m2[D:]
    zh = jnp.zeros((DH, D), jnp.float32)
    w2ae = jnp.concatenate([w2h[:DH], zh])
    w2be = jnp.concatenate([w2h[DH:], zh])
    w2ao = jnp.concatenate([zh, w2h[:DH]])
    w2bo = jnp.concatenate([zh, w2h[DH:]])
    za = jnp.zeros((DA, D), jnp.float32)
    w2Ae = jnp.concatenate([za, w2ea, za, za])
    w2Ao = jnp.concatenate([za, za, za, w2ea])
    b2 = b_m2.reshape(1, D)

    m_lo, m_hi = _edge_mlp(E, D, DA, 2000)(
        gd, gs, additional_message_features, edge_attr,
        additional_message_features, edge_attr, w1a, w1b, b1a, b1b,
        w2ae, w2be, w2ao, w2bo, w2Ae, w2Ao, b2)

    npad = -(-N // (8 * _NS)) * (8 * _NS)
    parts = _sc_scatter(N, E, D, 2)(
        m_lo, m_hi, dst, jnp.zeros((npad, D), jnp.float32))
    p = parts.reshape(_NC, npad, D)
    out = _update(N, D, DA, 2000)(
        x, p, additional_node_features, node_attr,
        W_u1, b_u1.reshape(1, D), W_u2, b_u2.reshape(1, D))
    return out


# speedup vs baseline: 1.3402x; 1.3386x over previous
import functools

import jax
import jax.numpy as jnp
from jax import lax
from jax.experimental import pallas as pl
from jax.experimental.pallas import tpu as pltpu
from jax.experimental.pallas import tpu_sc as plsc

_NC = 2
_NS = 16
_CH = 128
_NB = 3
_SNB = 2


def _pack_bf16_pair(m, DP):
    u = jax.lax.bitcast_convert_type(m, jnp.uint32)
    rnd = (u + ((u >> 16) & jnp.uint32(1)) + jnp.uint32(0x7FFF)) \
        & jnp.uint32(0xFFFF0000)
    packed = rnd[:, :DP] | (rnd[:, DP:] >> 16)
    return jax.lax.bitcast_convert_type(packed, jnp.int32)


def _unpack_bf16_pair(p):
    u = jax.lax.bitcast_convert_type(p, jnp.uint32)
    a = jax.lax.bitcast_convert_type(u & jnp.uint32(0xFFFF0000), jnp.float32)
    b = jax.lax.bitcast_convert_type(u << 16, jnp.float32)
    return a, b


def _proj_body(x_ref, wd_ref, ws_ref, pd_ref, ps_ref):
    xb = x_ref[...]
    dp = pd_ref.shape[-1]
    pd_ref[...] = _pack_bf16_pair(
        jnp.dot(xb, wd_ref[...], preferred_element_type=jnp.float32), dp)
    ps_ref[...] = _pack_bf16_pair(
        jnp.dot(xb, ws_ref[...], preferred_element_type=jnp.float32), dp)


def _proj(N, D, BN):
    return pl.pallas_call(
        _proj_body,
        grid=(N // BN,),
        in_specs=[
            pl.BlockSpec((BN, D), lambda i: (i, 0)),
            pl.BlockSpec((D, D), lambda i: (0, 0)),
            pl.BlockSpec((D, D), lambda i: (0, 0)),
        ],
        out_specs=[
            pl.BlockSpec((BN, D // 2), lambda i: (i, 0)),
            pl.BlockSpec((BN, D // 2), lambda i: (i, 0)),
        ],
        out_shape=[
            jax.ShapeDtypeStruct((N, D // 2), jnp.int32),
            jax.ShapeDtypeStruct((N, D // 2), jnp.int32),
        ],
    )


def _sc_gather(N, D, ES, ebase):
    DP = D // 2
    nw = _NC * _NS
    ew = ES // nw
    assert ES % nw == 0 and ew % 8 == 0 and ebase % 8 == 0 and ew >= _CH
    nch = -(-ew // _CH)

    mesh = plsc.VectorSubcoreMesh(core_axis_name="c", subcore_axis_name="s")

    @functools.partial(
        pl.kernel,
        mesh=mesh,
        compiler_params=pltpu.CompilerParams(use_tc_tiling_on_sc=False),
        out_type=[
            jax.ShapeDtypeStruct((ES, DP), jnp.int32),
            jax.ShapeDtypeStruct((ES, DP), jnp.int32),
        ],
        scratch_types=(
            [pltpu.VMEM((_CH,), jnp.int32) for _ in range(2 * _NB)]
            + [pltpu.VMEM((_CH, DP), jnp.int32) for _ in range(2 * _NB)]
            + [pltpu.SemaphoreType.DMA for _ in range(2 * _NB)]
        ),
    )
    def gather_k(pd_hbm, ps_hbm, dst_hbm, src_hbm, gd_hbm, gs_hbm, *scr):
        c = lax.axis_index("c")
        s = lax.axis_index("s")
        wid = s * _NC + c
        base_w = wid * ew
        idxs = scr[:2 * _NB]
        rows = scr[2 * _NB:4 * _NB]
        sems = scr[4 * _NB:]
        bufs = tuple(
            (idxs[2 * b], idxs[2 * b + 1], rows[2 * b], rows[2 * b + 1],
             sems[2 * b], sems[2 * b + 1])
            for b in range(_NB))
        la = _NB - 1

        def off(ch):
            return base_w + jnp.minimum(ch * _CH, ew - _CH)

        def fire(ch, b):
            dstv, srcv, rdv, rsv, gsem, _ = bufs[b]
            o = off(ch)
            pltpu.sync_copy(dst_hbm.at[pl.ds(ebase + o, _CH)], dstv)
            pltpu.sync_copy(src_hbm.at[pl.ds(ebase + o, _CH)], srcv)
            pltpu.async_copy(pd_hbm.at[dstv], rdv, gsem)
            pltpu.async_copy(ps_hbm.at[srcv], rsv, gsem)

        def drain_and_write(ch, b):
            dstv, srcv, rdv, rsv, gsem, wsem = bufs[b]
            o = off(ch)
            pltpu.make_async_copy(pd_hbm.at[dstv], rdv, gsem).wait()
            pltpu.make_async_copy(ps_hbm.at[srcv], rsv, gsem).wait()
            pltpu.async_copy(rdv, gd_hbm.at[pl.ds(o, _CH)], wsem)
            pltpu.async_copy(rsv, gs_hbm.at[pl.ds(o, _CH)], wsem)

        def wait_writes(ch, b):
            _, _, rdv, rsv, _, wsem = bufs[b]
            o = off(ch)
            pltpu.make_async_copy(rdv, gd_hbm.at[pl.ds(o, _CH)], wsem).wait()
            pltpu.make_async_copy(rsv, gs_hbm.at[pl.ds(o, _CH)], wsem).wait()

        for p in range(min(la, nch)):
            fire(p, p)

        @pl.loop(0, _NB * (-(-nch // _NB)), step=_NB)
        def _blk(i):
            for b in range(_NB):
                ch = i + b
                nxt = ch + la
                fb = (b + la) % _NB

                @pl.when(nxt < nch)
                def _():
                    @pl.when(nxt >= _NB)
                    def _():
                        wait_writes(nxt - _NB, fb)
                    fire(nxt, fb)

                @pl.when(ch < nch)
                def _():
                    drain_and_write(ch, b)

        for q in range(max(0, nch - _NB), nch):
            wait_writes(q, q % _NB)

    return gather_k


def _edge_body(gd_ref, gs_ref, ap_ref,
               w1a_ref, w1b_ref, b1a_ref, b1b_ref,
               w2ae_ref, w2be_ref, w2ao_ref, w2bo_ref, w2Ae_ref, w2Ao_ref,
               b2_ref, mev_ref, mod_ref):
    ad, bd = _unpack_bf16_pair(gd_ref[...])
    asrc, bsrc = _unpack_bf16_pair(gs_ref[...])
    ap = ap_ref[...]

    def mm(x, w_ref):
        return jnp.dot(x, w_ref[...], preferred_element_type=jnp.float32)

    ha = jax.nn.silu(ad + asrc + mm(ap, w1a_ref) + b1a_ref[...])
    hb = jax.nn.silu(bd + bsrc + mm(ap, w1b_ref) + b1b_ref[...])
    mev_ref[...] = jax.nn.silu(
        mm(ha, w2ae_ref) + mm(hb, w2be_ref) + mm(ap, w2Ae_ref)
        + b2_ref[...])
    mod_ref[...] = jax.nn.silu(
        mm(ha, w2ao_ref) + mm(hb, w2bo_ref) + mm(ap, w2Ao_ref)
        + b2_ref[...])


def _edge_mlp(ES, D, DA, BE):
    BE2 = BE // 2
    hoff = (ES // 2) // BE2
    full = lambda shape: pl.BlockSpec(shape, lambda i: (0, 0))
    return pl.pallas_call(
        _edge_body,
        grid=(ES // BE,),
        in_specs=[
            pl.BlockSpec((BE2, D), lambda i: (i, 0)),
            pl.BlockSpec((BE2, D), lambda i: (i, 0)),
            pl.BlockSpec((BE2, 4 * DA), lambda i: (i, 0)),
            full((4 * DA, D)), full((4 * DA, D)),
            full((1, D)), full((1, D)),
            full((D, D)), full((D, D)),
            full((D, D)), full((D, D)),
            full((4 * DA, D)), full((4 * DA, D)),
            full((1, D)),
        ],
        out_specs=[
            pl.BlockSpec((BE2, D), lambda i: (i, 0)),
            pl.BlockSpec((BE2, D), lambda i: (i, 0)),
        ],
        out_shape=[
            jax.ShapeDtypeStruct((ES // 2, D), jnp.float32),
            jax.ShapeDtypeStruct((ES // 2, D), jnp.float32),
        ],
    )


def _sc_scatter(N, E, D, K):
    nw = _NC * _NS
    ES = E // K
    ews = ES // nw
    ch_s = 120
    assert E % K == 0 and ES % nw == 0 and ews % 8 == 0 and ch_s % 8 == 0
    nfull = ews // ch_s
    tail = ews - nfull * ch_s
    assert tail % 8 == 0
    npad = -(-N // (8 * _NS)) * (8 * _NS)
    rps = npad // _NS

    mesh = plsc.VectorSubcoreMesh(core_axis_name="c", subcore_axis_name="s")

    scratch = (
        [pltpu.VMEM((ch_s,), jnp.int32) for _ in range(_SNB)]
        + [pltpu.VMEM((ch_s, D), jnp.float32) for _ in range(_SNB)]
        + [pltpu.SemaphoreType.DMA for _ in range(2 * _SNB)]
        + [pltpu.VMEM_SHARED((npad, D), jnp.float32)]
    )
    if tail:
        scratch += [
            pltpu.VMEM((tail,), jnp.int32),
            pltpu.VMEM((tail, D), jnp.float32),
        ]

    @functools.partial(
        pl.kernel,
        mesh=mesh,
        out_type=jax.ShapeDtypeStruct((_NC * npad, D), jnp.float32),
        scratch_types=scratch,
    )
    def scatter_k(*args):
        ms = args[:K]
        dst_hbm, zeros_hbm, out_hbm = args[K:K + 3]
        scr = args[K + 3:]
        c = lax.axis_index("c")
        s = lax.axis_index("s")
        wid = s * _NC + c
        base_w = wid * ews
        idxs = scr[:_SNB]
        mrows = scr[_SNB:2 * _SNB]
        sems = scr[2 * _SNB:4 * _SNB]
        acc = scr[4 * _SNB]
        tl = scr[4 * _SNB + 1:]
        bufs = tuple((idxs[b], mrows[b], sems[2 * b], sems[2 * b + 1])
                     for b in range(_SNB))
        la = _SNB - 1

        pltpu.sync_copy(zeros_hbm.at[pl.ds(s * rps, rps)],
                        acc.at[pl.ds(s * rps, rps)])
        plsc.subcore_barrier()

        def fire_loads(m_hbm, sbase, ch, b):
            dstv, mv, lsem, _ = bufs[b]
            o = base_w + ch * ch_s
            pltpu.async_copy(dst_hbm.at[pl.ds(sbase + o, ch_s)], dstv, lsem)
            pltpu.async_copy(m_hbm.at[pl.ds(o, ch_s)], mv, lsem)

        def fire_scatter(m_hbm, sbase, ch, b):
            dstv, mv, lsem, ssem = bufs[b]
            o = base_w + ch * ch_s
            pltpu.make_async_copy(
                dst_hbm.at[pl.ds(sbase + o, ch_s)], dstv, lsem).wait()
            pltpu.make_async_copy(m_hbm.at[pl.ds(o, ch_s)], mv, lsem).wait()
            pltpu.async_copy(mv, acc.at[dstv], ssem, add=True)

        def wait_scatter(b):
            dstv, mv, _, ssem = bufs[b]
            pltpu.make_async_copy(mv, acc.at[dstv], ssem).wait()

        for k in range(K):
            m_hbm = ms[k]
            sbase = k * ES
            for p in range(min(la, nfull)):
                fire_loads(m_hbm, sbase, p, p)

            @pl.loop(0, _SNB * (-(-nfull // _SNB)), step=_SNB)
            def _blk(i):
                for b in range(_SNB):
                    ch = i + b
                    nxt = ch + la
                    fb = (b + la) % _SNB

                    @pl.when(nxt < nfull)
                    def _():
                        @pl.when(nxt >= _SNB)
                        def _():
                            wait_scatter(fb)
                        fire_loads(m_hbm, sbase, nxt, fb)

                    @pl.when(ch < nfull)
                    def _():
                        fire_scatter(m_hbm, sbase, ch, b)

            for q in range(max(0, nfull - _SNB), nfull):
                wait_scatter(q % _SNB)

            if tail:
                dstv_t, mv_t = tl
                o = base_w + nfull * ch_s
                pltpu.sync_copy(dst_hbm.at[pl.ds(sbase + o, tail)], dstv_t)
                pltpu.sync_copy(m_hbm.at[pl.ds(o, tail)], mv_t)
                pltpu.sync_copy(mv_t, acc.at[dstv_t], add=True)

        plsc.subcore_barrier()
        pltpu.sync_copy(acc.at[pl.ds(s * rps, rps)],
                        out_hbm.at[pl.ds(c * npad + s * rps, rps)])

    return scatter_k


def _update_body(x_ref, p_ref, anf_ref, na_ref, wu1_ref, bu1_ref,
                 wu2_ref, bu2_ref, o_ref):
    na = na_ref[...]
    agg = p_ref[0] + p_ref[1]
    u_in = jnp.concatenate([x_ref[...], agg, anf_ref[...], na], axis=-1)
    u = jax.nn.silu(jnp.dot(u_in, wu1_ref[...],
                            preferred_element_type=jnp.float32) + bu1_ref[...])
    o_ref[...] = jnp.dot(jnp.concatenate([u, na], axis=-1), wu2_ref[...],
                         preferred_element_type=jnp.float32) + bu2_ref[...]


def _update(N, D, DA, BN):
    return pl.pallas_call(
        _update_body,
        grid=(N // BN,),
        in_specs=[
            pl.BlockSpec((BN, D), lambda i: (i, 0)),
            pl.BlockSpec((_NC, BN, D), lambda i: (0, i, 0)),
            pl.BlockSpec((BN, DA), lambda i: (i, 0)),
            pl.BlockSpec((BN, DA), lambda i: (i, 0)),
            pl.BlockSpec((2 * D + 2 * DA, D), lambda i: (0, 0)),
            pl.BlockSpec((1, D), lambda i: (0, 0)),
            pl.BlockSpec((D + DA, D), lambda i: (0, 0)),
            pl.BlockSpec((1, D), lambda i: (0, 0)),
        ],
        out_specs=pl.BlockSpec((BN, D), lambda i: (i, 0)),
        out_shape=jax.ShapeDtypeStruct((N, D), jnp.float32),
    )


def kernel(x, edge_index, edge_attr, node_attr, batch,
           additional_message_features, additional_node_features,
           W_m1, b_m1, W_m2, b_m2, W_u1, b_u1, W_u2, b_u2):
    del batch
    N, D = x.shape
    E, DA = edge_attr.shape
    DH = D // 2
    src = edge_index[0]
    dst = edge_index[1]

    pd, ps = _proj(N, D, 2000)(x, W_m1[:D], W_m1[D:2 * D])
    gd, gs = _sc_gather(N, D, E, 0)(pd, ps, dst, src)
    gd = gd.reshape(E // 2, D)
    gs = gs.reshape(E // 2, D)

    ap = jnp.concatenate([additional_message_features, edge_attr],
                         axis=-1).reshape(E // 2, 4 * DA)
    w1t = W_m1[2 * D:]
    z = jnp.zeros((2 * DA, DH), jnp.float32)
    w1a = jnp.block([[w1t[:, :DH], z], [z, w1t[:, :DH]]])
    w1b = jnp.block([[w1t[:, DH:], z], [z, w1t[:, DH:]]])
    b1a = jnp.tile(b_m1[:DH], 2).reshape(1, D)
    b1b = jnp.tile(b_m1[DH:], 2).reshape(1, D)
    w2h = W_m2[:D]
    w2ea = W_m2[D:]
    zh = jnp.zeros((DH, D), jnp.float32)
    w2ae = jnp.concatenate([w2h[:DH], zh])
    w2be = jnp.concatenate([w2h[DH:], zh])
    w2ao = jnp.concatenate([zh, w2h[:DH]])
    w2bo = jnp.concatenate([zh, w2h[DH:]])
    za = jnp.zeros((DA, D), jnp.float32)
    w2Ae = jnp.concatenate([za, w2ea, za, za])
    w2Ao = jnp.concatenate([za, za, za, w2ea])
    b2 = b_m2.reshape(1, D)

    m_ev, m_od = _edge_mlp(E, D, DA, 2000)(
        gd, gs, ap, w1a, w1b, b1a, b1b,
        w2ae, w2be, w2ao, w2bo, w2Ae, w2Ao, b2)

    dst_perm = jnp.concatenate([dst[0::2], dst[1::2]])
    npad = -(-N // (8 * _NS)) * (8 * _NS)
    parts = _sc_scatter(N, E, D, 2)(
        m_ev, m_od, dst_perm, jnp.zeros((npad, D), jnp.float32))
    p = parts.reshape(_NC, npad, D)
    out = _update(N, D, DA, 2000)(
        x, p, additional_node_features, node_attr,
        W_u1, b_u1.reshape(1, D), W_u2, b_u2.reshape(1, D))
    return out
